# baseline (speedup 1.0000x reference)
import functools

import jax
import jax.numpy as jnp
from jax import lax
from jax.experimental import pallas as pl
from jax.experimental.pallas import tpu as pltpu
from jax.experimental.pallas import tpu_sc as plsc

_NC, _NS = 2, 16
_NW = _NC * _NS
_L = 16


def _sc_gather(tables, x):
    f, v, d = tables.shape
    bsz = x.shape[0]
    samples_pw = bsz // _NW
    cs = 128
    n_chunks = samples_pw // cs

    mesh = plsc.VectorSubcoreMesh(core_axis_name="c", subcore_axis_name="s")

    def body(tab_hbm, x_hbm, out_hbm, x_v, idx_v, gat_v, sem):
        wid = lax.axis_index("s") * _NC + lax.axis_index("c")

        def step(i, carry):
            s0 = wid * samples_pw + i * cs
            pltpu.sync_copy(x_hbm.at[pl.ds(s0, cs)], x_v)

            def feat(fi, carry2):
                def build(j, carry3):
                    r = j * _L + lax.iota(jnp.int32, _L)
                    xv = plsc.load_gather(x_v, [r, jnp.full((_L,), fi, jnp.int32)])
                    idx_v[pl.ds(j * _L, _L)] = xv
                    return carry3

                lax.fori_loop(0, cs // _L, build, 0)
                pltpu.async_copy(tab_hbm.at[fi].at[idx_v], gat_v, sem).wait()
                pltpu.sync_copy(gat_v, out_hbm.at[pl.ds(s0, cs), fi, :])
                return carry2

            lax.fori_loop(0, f, feat, 0)
            return carry

        lax.fori_loop(0, n_chunks, step, 0)

    fn = pl.kernel(
        body,
        name="sc_embed_gather",
        out_type=jax.ShapeDtypeStruct((bsz, f, d), jnp.float32),
        mesh=mesh,
        scratch_types=[
            pltpu.VMEM((cs, f), jnp.int32),
            pltpu.VMEM((cs,), jnp.int32),
            pltpu.VMEM((cs, d), jnp.float32),
            pltpu.SemaphoreType.DMA,
        ],
        compiler_params=pltpu.CompilerParams(
            use_tc_tiling_on_sc=False, needs_layout_passes=False
        ),
    )
    return fn(tables, x)


def _tc_project(concat, wt, b):
    bsz, k = concat.shape
    o = wt.shape[1]
    bm = 2048

    def body(a_ref, w_ref, b_ref, o_ref):
        o_ref[...] = (
            jnp.dot(a_ref[...], w_ref[...], preferred_element_type=jnp.float32)
            + b_ref[...]
        )

    return pl.pallas_call(
        body,
        grid=(bsz // bm,),
        in_specs=[
            pl.BlockSpec((bm, k), lambda i: (i, 0)),
            pl.BlockSpec((k, o), lambda i: (0, 0)),
            pl.BlockSpec((1, o), lambda i: (0, 0)),
        ],
        out_specs=pl.BlockSpec((bm, o), lambda i: (i, 0)),
        out_shape=jax.ShapeDtypeStruct((bsz, o), jnp.float32),
    )(concat, wt, b.reshape(1, o))


def kernel(x, tables, W, b):
    bsz, f = x.shape
    _, v, d = tables.shape
    emb = _sc_gather(tables, x)
    return _tc_project(emb.reshape(bsz, f * d), W.T, b)


# speedup vs baseline: 12.5623x; 12.5623x over previous
import jax
import jax.numpy as jnp
from jax import lax
from jax.experimental import pallas as pl
from jax.experimental.pallas import tpu as pltpu
from jax.experimental.pallas import tpu_sc as plsc

_NC, _NS = 2, 16
_NW = _NC * _NS
_L = 16


def _sc_gather_t(tables_t, x_t):
    f, d, v = tables_t.shape
    bsz = x_t.shape[1]
    n_rows = f * d
    rpw = n_rows // _NW
    bc = 4096
    n_bc = bsz // bc

    mesh = plsc.VectorSubcoreMesh(core_axis_name="c", subcore_axis_name="s")

    def body(tab_hbm, x_hbm, val_hbm, row_v, xb_v, va_v, vb_v, semr, semx, semv):
        wid = lax.axis_index("s") * _NC + lax.axis_index("c")
        j0 = wid * rpw

        def start_row(jj):
            fi2 = lax.shift_right_logical(jj, 4)
            dd2 = lax.bitwise_and(jj, d - 1)
            pltpu.async_copy(tab_hbm.at[fi2, dd2, :], row_v, semr)

        start_row(j0)

        def row_task(r, prev_fi):
            j = j0 + r
            fi = lax.shift_right_logical(j, 4)
            dd = lax.bitwise_and(j, d - 1)

            @pl.when(fi != prev_fi)
            def _load_x():
                pltpu.async_copy(x_hbm.at[fi, :], xb_v, semx).wait()

            pltpu.make_async_copy(tab_hbm.at[fi, dd, :], row_v, semr).wait()

            bufs = (va_v, vb_v)
            for c in range(n_bc):
                buf = bufs[c % 2]
                if c >= 2:
                    pltpu.make_async_copy(
                        buf, val_hbm.at[j, pl.ds(0, bc)], semv
                    ).wait()
                else:
                    @pl.when(r > 0)
                    def _w():
                        pltpu.make_async_copy(
                            buf, val_hbm.at[j, pl.ds(0, bc)], semv
                        ).wait()

                b0 = c * bc

                def gat(k, carry3):
                    base = b0 + k * (8 * _L)
                    idxs = [xb_v[pl.ds(base + t * _L, _L)] for t in range(8)]
                    gs = [plsc.load_gather(row_v, [ix]) for ix in idxs]
                    for t in range(8):
                        buf[pl.ds(k * (8 * _L) + t * _L, _L)] = gs[t]
                    return carry3

                lax.fori_loop(0, bc // (8 * _L), gat, 0)
                pltpu.async_copy(buf, val_hbm.at[j, pl.ds(b0, bc)], semv)

            @pl.when(r + 1 < rpw)
            def _prefetch():
                start_row(j + 1)

            return fi

        lax.fori_loop(0, rpw, row_task, jnp.int32(-1))
        pltpu.make_async_copy(va_v, val_hbm.at[0, pl.ds(0, bc)], semv).wait()
        pltpu.make_async_copy(vb_v, val_hbm.at[0, pl.ds(0, bc)], semv).wait()

    fn = pl.kernel(
        body,
        name="sc_embed_gather",
        out_type=jax.ShapeDtypeStruct((n_rows, bsz), jnp.float32),
        mesh=mesh,
        scratch_types=[
            pltpu.VMEM((v,), jnp.float32),
            pltpu.VMEM((bsz,), jnp.int32),
            pltpu.VMEM((bc,), jnp.float32),
            pltpu.VMEM((bc,), jnp.float32),
            pltpu.SemaphoreType.DMA,
            pltpu.SemaphoreType.DMA,
            pltpu.SemaphoreType.DMA,
        ],
        compiler_params=pltpu.CompilerParams(
            use_tc_tiling_on_sc=True, needs_layout_passes=False
        ),
    )
    return fn(tables_t, x_t)


def _tc_project_t(val, w, b):
    k, bsz = val.shape
    o = w.shape[0]
    bm = 4096

    def body(w_ref, v_ref, b_ref, o_ref):
        o_ref[...] = (
            jax.lax.dot_general(
                w_ref[...], v_ref[...],
                (((1,), (0,)), ((), ())),
                preferred_element_type=jnp.float32,
            )
            + b_ref[...]
        )

    return pl.pallas_call(
        body,
        grid=(bsz // bm,),
        in_specs=[
            pl.BlockSpec((o, k), lambda i: (0, 0)),
            pl.BlockSpec((k, bm), lambda i: (0, i)),
            pl.BlockSpec((o, 1), lambda i: (0, 0)),
        ],
        out_specs=pl.BlockSpec((o, bm), lambda i: (0, i)),
        out_shape=jax.ShapeDtypeStruct((o, bsz), jnp.float32),
    )(w, val, b.reshape(o, 1))


def kernel(x, tables, W, b):
    tables_t = jnp.transpose(tables, (0, 2, 1))
    x_t = x.T
    val = _sc_gather_t(tables_t, x_t)
    out_t = _tc_project_t(val, W, b)
    return out_t.T
